# Initial kernel scaffold; baseline (speedup 1.0000x reference)
import jax
import jax.numpy as jnp
from jax.experimental import pallas as pl


def kernel(x, edge_index, order_attn, W_self1, W_neigh1, b1, bn1_g, bn1_b, W_self2, W_neigh2, b2, bn2_g, bn2_b, W_self3, W_neigh3, b3):
    raise NotImplementedError("write your pallas kernel here")


# speedup vs baseline: 4.6227x; 4.6227x over previous
import functools

import jax
import jax.numpy as jnp
from jax import lax
from jax.experimental import pallas as pl
from jax.experimental.pallas import tpu as pltpu
from jax.experimental.pallas import tpu_sc as plsc

N = 10000
D = 128
E = 320000

NC = 2
NS = 16
NW = NC * NS
EPW = E // NW
CHUNK = 80
NCHUNK = EPW // CHUNK
NP = 10240
RPT = NP // NS

_mesh = plsc.VectorSubcoreMesh(core_axis_name="c", subcore_axis_name="s")

_f32 = jnp.float32


@functools.partial(
    pl.kernel, mesh=_mesh,
    out_type=jax.ShapeDtypeStruct((NC * NP, D), _f32),
    scratch_types=[
        pltpu.VMEM((CHUNK,), jnp.int32),
        pltpu.VMEM((CHUNK,), jnp.int32),
        pltpu.VMEM((CHUNK, D), _f32),
        pltpu.VMEM_SHARED((NP, D), _f32),
        pltpu.SemaphoreType.DMA,
    ],
)
def _agg(table, src, dst, z_rows, acc_out, src_v, dst_v, rows_v, acc_sh, sem):
    cid = lax.axis_index("c")
    sid = lax.axis_index("s")
    wid = sid * NC + cid
    row0 = pl.multiple_of(sid * RPT, 8)
    pltpu.sync_copy(z_rows, acc_sh.at[pl.ds(row0, RPT)])
    plsc.subcore_barrier()

    def step(i, carry):
        base = pl.multiple_of(wid * EPW + i * CHUNK, 8)
        pltpu.sync_copy(src.at[pl.ds(base, CHUNK)], src_v)
        pltpu.sync_copy(dst.at[pl.ds(base, CHUNK)], dst_v)
        pltpu.async_copy(table.at[src_v], rows_v, sem).wait()
        pltpu.sync_copy(rows_v, acc_sh.at[dst_v], add=True)
        return carry

    lax.fori_loop(0, NCHUNK, step, 0)
    plsc.subcore_barrier()
    out_row0 = pl.multiple_of(cid * NP + sid * RPT, 8)
    pltpu.sync_copy(acc_sh.at[pl.ds(row0, RPT)], acc_out.at[pl.ds(out_row0, RPT)])


@functools.partial(
    pl.kernel, mesh=_mesh,
    out_type=jax.ShapeDtypeStruct((NC * NP, D), _f32),
    scratch_types=[
        pltpu.VMEM((CHUNK,), jnp.int32),
        pltpu.VMEM((CHUNK, D), _f32),
        pltpu.VMEM_SHARED((NP, D), _f32),
        pltpu.SemaphoreType.DMA,
    ],
)
def _deg(dst, z_rows, ones_h, deg_out, dst_v, ones_v, deg_sh, sem):
    cid = lax.axis_index("c")
    sid = lax.axis_index("s")
    wid = sid * NC + cid
    row0 = pl.multiple_of(sid * RPT, 8)
    pltpu.sync_copy(z_rows, deg_sh.at[pl.ds(row0, RPT)])
    pltpu.sync_copy(ones_h, ones_v)
    plsc.subcore_barrier()

    def step(i, carry):
        base = pl.multiple_of(wid * EPW + i * CHUNK, 8)
        pltpu.sync_copy(dst.at[pl.ds(base, CHUNK)], dst_v)
        pltpu.sync_copy(ones_v, deg_sh.at[dst_v], add=True)
        return carry

    lax.fori_loop(0, NCHUNK, step, 0)
    plsc.subcore_barrier()
    out_row0 = pl.multiple_of(cid * NP + sid * RPT, 8)
    pltpu.sync_copy(deg_sh.at[pl.ds(row0, RPT)], deg_out.at[pl.ds(out_row0, RPT)])


def _tc_in(x_ref, ws_ref, wn_ref, b_ref, hs_ref, hn_ref):
    x = x_ref[...]
    hs_ref[...] = jnp.dot(x, ws_ref[...], preferred_element_type=_f32) + b_ref[...]
    hn_ref[...] = jnp.dot(x, wn_ref[...], preferred_element_type=_f32)


def _tc_mid(hs_ref, acc_ref, degp_ref, g_ref, bb_ref, ws_ref, wn_ref, b_ref,
            hs2_ref, hn2_ref):
    deg = degp_ref[0, :, :1] + degp_ref[1, :, :1]
    t = hs_ref[...] + (acc_ref[0] + acc_ref[1]) / jnp.maximum(deg, 1.0)
    mu = jnp.mean(t, axis=0, keepdims=True)
    var = jnp.mean((t - mu) ** 2, axis=0, keepdims=True)
    z = jnp.maximum((t - mu) / jnp.sqrt(var + 1e-5) * g_ref[...] + bb_ref[...],
                    0.0)
    hs2_ref[...] = jnp.dot(z, ws_ref[...], preferred_element_type=_f32) + b_ref[...]
    hn2_ref[...] = jnp.dot(z, wn_ref[...], preferred_element_type=_f32)


def _tc_fin(hs_ref, acc_ref, degp_ref, o_ref):
    deg = degp_ref[0, :, :1] + degp_ref[1, :, :1]
    t = hs_ref[...] + (acc_ref[0] + acc_ref[1]) / jnp.maximum(deg, 1.0)
    m = jnp.max(t, axis=1, keepdims=True)
    s = t - m
    o_ref[...] = s - jnp.log(jnp.sum(jnp.exp(s), axis=1, keepdims=True))


_nd = jax.ShapeDtypeStruct((N, D), _f32)
_tc_in_call = pl.pallas_call(_tc_in, out_shape=[_nd, _nd])
_tc_mid_call = pl.pallas_call(_tc_mid, out_shape=[_nd, _nd])
_tc_fin_call = pl.pallas_call(_tc_fin, out_shape=_nd)


def kernel(x, edge_index, order_attn, W_self1, W_neigh1, b1, bn1_g, bn1_b,
           W_self2, W_neigh2, b2, bn2_g, bn2_b, W_self3, W_neigh3, b3):
    src = edge_index[0]
    dst = edge_index[1]
    z_rows = jnp.zeros((RPT, D), _f32)
    ones_h = jnp.ones((CHUNK, D), _f32)

    def unpad(a):
        return a.reshape(NC, NP, D)[:, :N]

    hs1, hn1 = _tc_in_call(x, W_self1, W_neigh1, b1.reshape(1, D))
    degp = unpad(_deg(dst, z_rows, ones_h))
    acc1 = unpad(_agg(hn1, src, dst, z_rows))
    hs2, hn2 = _tc_mid_call(hs1, acc1, degp, bn1_g.reshape(1, D),
                            bn1_b.reshape(1, D), W_self2, W_neigh2,
                            b2.reshape(1, D))
    acc2 = unpad(_agg(hn2, src, dst, z_rows))
    hs3, hn3 = _tc_mid_call(hs2, acc2, degp, bn2_g.reshape(1, D),
                            bn2_b.reshape(1, D), W_self3, W_neigh3,
                            b3.reshape(1, D))
    acc3 = unpad(_agg(hn3, src, dst, z_rows))
    return _tc_fin_call(hs3, acc3, degp)


# speedup vs baseline: 8.6270x; 1.8662x over previous
import functools

import jax
import jax.numpy as jnp
from jax import lax
from jax.experimental import pallas as pl
from jax.experimental.pallas import tpu as pltpu
from jax.experimental.pallas import tpu_sc as plsc

N = 10000
D = 128
E = 320000

NC = 2
NS = 16
NW = NC * NS
EPW = E // NW
CH = 125
NCH = EPW // CH
NP = 10240
RPT = NP // NS

_mesh = plsc.VectorSubcoreMesh(core_axis_name="c", subcore_axis_name="s")

_f32 = jnp.float32

_sc_agg = [
    pltpu.VMEM((NCH, CH), jnp.int32),
    pltpu.VMEM((CH,), jnp.int32),
    pltpu.VMEM((CH,), jnp.int32),
    pltpu.VMEM((CH, D), _f32),
    pltpu.VMEM((CH, D), _f32),
    pltpu.VMEM_SHARED((NP, D), _f32),
] + [pltpu.SemaphoreType.DMA] * 6


@functools.partial(pl.kernel, mesh=_mesh,
                   out_type=jax.ShapeDtypeStruct((NC * NP, D), _f32),
                   scratch_types=_sc_agg)
def _agg(table, src2, dst2, z_rows, acc_out, dst_v, i0, i1, b0, b1,
         acc_sh, is0, is1, gs0, gs1, ss0, ss1):
    isl = [i0, i1]
    bufs = [b0, b1]
    isem = [is0, is1]
    gsem = [gs0, gs1]
    ssem = [ss0, ss1]
    cid = lax.axis_index("c")
    sid = lax.axis_index("s")
    wid = sid * NC + cid
    row0 = pl.multiple_of(sid * RPT, 8)
    crow = pl.multiple_of(wid * NCH, 8)
    pltpu.sync_copy(z_rows, acc_sh.at[pl.ds(row0, RPT)])
    pltpu.sync_copy(dst2.at[pl.ds(crow, NCH)], dst_v)
    plsc.subcore_barrier()

    def idx_load(k, c):
        pltpu.async_copy(src2.at[crow + c], isl[k], isem[k])

    def idx_wait(k):
        pltpu.make_async_copy(src2.at[crow], isl[k], isem[k]).wait()

    def gather(k):
        pltpu.async_copy(table.at[isl[k]], bufs[k], gsem[k])

    def gather_wait(k):
        pltpu.make_async_copy(table.at[isl[k]], bufs[k], gsem[k]).wait()

    def scatter(k, c):
        pltpu.async_copy(bufs[k], acc_sh.at[dst_v.at[c]], ssem[k], add=True)

    def scatter_wait(k):
        pltpu.make_async_copy(bufs[k], acc_sh.at[dst_v.at[0]], ssem[k]).wait()

    for k in range(2):
        idx_load(k, k)
    for k in range(2):
        idx_wait(k)
        gather(k)
    for k in range(2):
        gather_wait(k)
        scatter(k, k)
        idx_load(k, jnp.minimum(2 + k, NCH - 1))

    def round_body(r, carry):
        c0 = r * 2
        for k in range(2):
            scatter_wait(k)
            idx_wait(k)
            gather(k)
        for k in range(2):
            gather_wait(k)
            scatter(k, c0 + k)
            idx_load(k, jnp.minimum(c0 + 2 + k, NCH - 1))
        return carry

    lax.fori_loop(1, NCH // 2, round_body, 0)
    for k in range(2):
        scatter_wait(k)
        idx_wait(k)
    plsc.subcore_barrier()
    out_row0 = pl.multiple_of(cid * NP + sid * RPT, 8)
    pltpu.sync_copy(acc_sh.at[pl.ds(row0, RPT)], acc_out.at[pl.ds(out_row0, RPT)])


KS = 4

_sc_deg = [
    pltpu.VMEM((NCH, CH), jnp.int32),
    pltpu.VMEM((CH, D), _f32),
    pltpu.VMEM_SHARED((NP, D), _f32),
] + [pltpu.SemaphoreType.DMA] * KS


@functools.partial(pl.kernel, mesh=_mesh,
                   out_type=jax.ShapeDtypeStruct((NC * NP, D), _f32),
                   scratch_types=_sc_deg)
def _deg(dst2, z_rows, ones_h, deg_out, dst_v, ones_v, deg_sh, *ss):
    cid = lax.axis_index("c")
    sid = lax.axis_index("s")
    wid = sid * NC + cid
    row0 = pl.multiple_of(sid * RPT, 8)
    crow = pl.multiple_of(wid * NCH, 8)
    pltpu.sync_copy(z_rows, deg_sh.at[pl.ds(row0, RPT)])
    pltpu.sync_copy(ones_h, ones_v)
    pltpu.sync_copy(dst2.at[pl.ds(crow, NCH)], dst_v)
    plsc.subcore_barrier()

    for k in range(KS):
        pltpu.async_copy(ones_v, deg_sh.at[dst_v.at[k]], ss[k], add=True)

    def body(r, carry):
        c0 = (r + 1) * KS
        for k in range(KS):
            pltpu.make_async_copy(ones_v, deg_sh.at[dst_v.at[0]], ss[k]).wait()
            pltpu.async_copy(ones_v, deg_sh.at[dst_v.at[c0 + k]], ss[k],
                             add=True)
        return carry

    lax.fori_loop(0, NCH // KS - 1, body, 0)
    for k in range(KS):
        pltpu.make_async_copy(ones_v, deg_sh.at[dst_v.at[0]], ss[k]).wait()
    plsc.subcore_barrier()
    out_row0 = pl.multiple_of(cid * NP + sid * RPT, 8)
    pltpu.sync_copy(deg_sh.at[pl.ds(row0, RPT)], deg_out.at[pl.ds(out_row0, RPT)])


def _tc_in(x_ref, ws_ref, wn_ref, b_ref, hs_ref, hn_ref):
    x = x_ref[...]
    hs_ref[...] = jnp.dot(x, ws_ref[...], preferred_element_type=_f32) + b_ref[...]
    hn_ref[...] = jnp.dot(x, wn_ref[...], preferred_element_type=_f32)


def _tc_mid(hs_ref, acc_ref, degp_ref, g_ref, bb_ref, ws_ref, wn_ref, b_ref,
            hs2_ref, hn2_ref):
    deg = degp_ref[0, :, :1] + degp_ref[1, :, :1]
    t = hs_ref[...] + (acc_ref[0] + acc_ref[1]) / jnp.maximum(deg, 1.0)
    mu = jnp.mean(t, axis=0, keepdims=True)
    var = jnp.mean((t - mu) ** 2, axis=0, keepdims=True)
    z = jnp.maximum((t - mu) / jnp.sqrt(var + 1e-5) * g_ref[...] + bb_ref[...],
                    0.0)
    hs2_ref[...] = jnp.dot(z, ws_ref[...], preferred_element_type=_f32) + b_ref[...]
    hn2_ref[...] = jnp.dot(z, wn_ref[...], preferred_element_type=_f32)


def _tc_fin(hs_ref, acc_ref, degp_ref, o_ref):
    deg = degp_ref[0, :, :1] + degp_ref[1, :, :1]
    t = hs_ref[...] + (acc_ref[0] + acc_ref[1]) / jnp.maximum(deg, 1.0)
    m = jnp.max(t, axis=1, keepdims=True)
    s = t - m
    o_ref[...] = s - jnp.log(jnp.sum(jnp.exp(s), axis=1, keepdims=True))


_nd = jax.ShapeDtypeStruct((N, D), _f32)
_tc_in_call = pl.pallas_call(_tc_in, out_shape=[_nd, _nd])
_tc_mid_call = pl.pallas_call(_tc_mid, out_shape=[_nd, _nd])
_tc_fin_call = pl.pallas_call(_tc_fin, out_shape=_nd)


def kernel(x, edge_index, order_attn, W_self1, W_neigh1, b1, bn1_g, bn1_b,
           W_self2, W_neigh2, b2, bn2_g, bn2_b, W_self3, W_neigh3, b3):
    src2 = edge_index[0].reshape(NW * NCH, CH)
    dst2 = edge_index[1].reshape(NW * NCH, CH)
    z_rows = jnp.zeros((RPT, D), _f32)
    ones_h = jnp.ones((CH, D), _f32)

    def unpad(a):
        return a.reshape(NC, NP, D)[:, :N]

    hs1, hn1 = _tc_in_call(x, W_self1, W_neigh1, b1.reshape(1, D))
    degp = unpad(_deg(dst2, z_rows, ones_h))
    acc1 = unpad(_agg(hn1, src2, dst2, z_rows))
    hs2, hn2 = _tc_mid_call(hs1, acc1, degp, bn1_g.reshape(1, D),
                            bn1_b.reshape(1, D), W_self2, W_neigh2,
                            b2.reshape(1, D))
    acc2 = unpad(_agg(hn2, src2, dst2, z_rows))
    hs3, hn3 = _tc_mid_call(hs2, acc2, degp, bn2_g.reshape(1, D),
                            bn2_b.reshape(1, D), W_self3, W_neigh3,
                            b3.reshape(1, D))
    acc3 = unpad(_agg(hn3, src2, dst2, z_rows))
    return _tc_fin_call(hs3, acc3, degp)


# speedup vs baseline: 10.3452x; 1.1992x over previous
import functools

import jax
import jax.numpy as jnp
from jax import lax
from jax.experimental import pallas as pl
from jax.experimental.pallas import tpu as pltpu
from jax.experimental.pallas import tpu_sc as plsc

N = 10000
D = 128
E = 320000

NC = 2
NS = 16
NW = NC * NS
EPW = E // NW
CHA = 50
NCHA = EPW // CHA
SLOTS = 4
CHD = 125
NCHD = EPW // CHD
KS = 4
NP = 10240
RPT = NP // NS

_mesh = plsc.VectorSubcoreMesh(core_axis_name="c", subcore_axis_name="s")

_f32 = jnp.float32

_sc_agg = ([pltpu.VMEM((2, 2, CHA), jnp.int32) for _ in range(SLOTS)]
           + [pltpu.VMEM((CHA, D), _f32) for _ in range(SLOTS)]
           + [pltpu.VMEM_SHARED((NP, D), _f32)]
           + [pltpu.SemaphoreType.DMA] * (3 * SLOTS))


@functools.partial(pl.kernel, mesh=_mesh,
                   out_type=jax.ShapeDtypeStruct((NC * NP, D), _f32),
                   scratch_types=_sc_agg)
def _agg(table, sd2, z_rows, acc_out, *rest):
    isl = list(rest[:SLOTS])
    bufs = list(rest[SLOTS:2 * SLOTS])
    acc_sh = rest[2 * SLOTS]
    isem = list(rest[2 * SLOTS + 1:2 * SLOTS + 1 + SLOTS])
    gsem = list(rest[2 * SLOTS + 1 + SLOTS:2 * SLOTS + 1 + 2 * SLOTS])
    ssem = list(rest[2 * SLOTS + 1 + 2 * SLOTS:])
    cid = lax.axis_index("c")
    sid = lax.axis_index("s")
    wid = sid * NC + cid
    row0 = pl.multiple_of(sid * RPT, 8)
    crow = pl.multiple_of(wid * NCHA, 8)
    pltpu.sync_copy(z_rows, acc_sh.at[pl.ds(row0, RPT)])
    plsc.subcore_barrier()

    def idx_load(k, c, p):
        pltpu.async_copy(sd2.at[crow + c], isl[k].at[p], isem[k])

    def idx_wait(k):
        pltpu.make_async_copy(sd2.at[crow], isl[k].at[0], isem[k]).wait()

    def gather(k, p):
        pltpu.async_copy(table.at[isl[k].at[p, 0]], bufs[k], gsem[k])

    def gather_wait(k, p):
        pltpu.make_async_copy(table.at[isl[k].at[p, 0]], bufs[k],
                              gsem[k]).wait()

    def scatter(k, p):
        pltpu.async_copy(bufs[k], acc_sh.at[isl[k].at[p, 1]], ssem[k],
                         add=True)

    def scatter_wait(k):
        pltpu.make_async_copy(bufs[k], acc_sh.at[isl[k].at[0, 1]],
                              ssem[k]).wait()

    for k in range(SLOTS):
        idx_load(k, k, 0)
    for k in range(SLOTS):
        idx_wait(k)
        gather(k, 0)
    for k in range(SLOTS):
        gather_wait(k, 0)
        scatter(k, 0)
        idx_load(k, jnp.minimum(SLOTS + k, NCHA - 1), 1)

    def round_body(r, carry):
        c0 = r * SLOTS
        p = r % 2
        for k in range(SLOTS):
            scatter_wait(k)
            idx_wait(k)
            gather(k, p)
        for k in range(SLOTS):
            gather_wait(k, p)
            scatter(k, p)
            idx_load(k, jnp.minimum(c0 + SLOTS + k, NCHA - 1), 1 - p)
        return carry

    lax.fori_loop(1, NCHA // SLOTS, round_body, 0)
    for k in range(SLOTS):
        scatter_wait(k)
        idx_wait(k)
    plsc.subcore_barrier()
    out_row0 = pl.multiple_of(cid * NP + sid * RPT, 8)
    pltpu.sync_copy(acc_sh.at[pl.ds(row0, RPT)], acc_out.at[pl.ds(out_row0, RPT)])


_sc_deg = [
    pltpu.VMEM((NCHD, CHD), jnp.int32),
    pltpu.VMEM((CHD, D), _f32),
    pltpu.VMEM_SHARED((NP, D), _f32),
] + [pltpu.SemaphoreType.DMA] * KS


@functools.partial(pl.kernel, mesh=_mesh,
                   out_type=jax.ShapeDtypeStruct((NC * NP, D), _f32),
                   scratch_types=_sc_deg)
def _deg(dst2, z_rows, ones_h, deg_out, dst_v, ones_v, deg_sh, *ss):
    cid = lax.axis_index("c")
    sid = lax.axis_index("s")
    wid = sid * NC + cid
    row0 = pl.multiple_of(sid * RPT, 8)
    crow = pl.multiple_of(wid * NCHD, 8)
    pltpu.sync_copy(z_rows, deg_sh.at[pl.ds(row0, RPT)])
    pltpu.sync_copy(ones_h, ones_v)
    pltpu.sync_copy(dst2.at[pl.ds(crow, NCHD)], dst_v)
    plsc.subcore_barrier()

    for k in range(KS):
        pltpu.async_copy(ones_v, deg_sh.at[dst_v.at[k]], ss[k], add=True)

    def body(r, carry):
        c0 = (r + 1) * KS
        for k in range(KS):
            pltpu.make_async_copy(ones_v, deg_sh.at[dst_v.at[0]], ss[k]).wait()
            pltpu.async_copy(ones_v, deg_sh.at[dst_v.at[c0 + k]], ss[k],
                             add=True)
        return carry

    lax.fori_loop(0, NCHD // KS - 1, body, 0)
    for k in range(KS):
        pltpu.make_async_copy(ones_v, deg_sh.at[dst_v.at[0]], ss[k]).wait()
    plsc.subcore_barrier()
    out_row0 = pl.multiple_of(cid * NP + sid * RPT, 8)
    pltpu.sync_copy(deg_sh.at[pl.ds(row0, RPT)], deg_out.at[pl.ds(out_row0, RPT)])


def _tc_in(x_ref, ws_ref, wn_ref, b_ref, hs_ref, hn_ref):
    x = x_ref[...]
    hs_ref[...] = jnp.dot(x, ws_ref[...], preferred_element_type=_f32) + b_ref[...]
    hn_ref[...] = jnp.dot(x, wn_ref[...], preferred_element_type=_f32)


def _tc_mid(hs_ref, acc_ref, degp_ref, g_ref, bb_ref, ws_ref, wn_ref, b_ref,
            hs2_ref, hn2_ref):
    deg = degp_ref[0, :, :1] + degp_ref[1, :, :1]
    a = acc_ref[0:N] + acc_ref[NP:NP + N]
    t = hs_ref[...] + a / jnp.maximum(deg, 1.0)
    mu = jnp.mean(t, axis=0, keepdims=True)
    var = jnp.mean((t - mu) ** 2, axis=0, keepdims=True)
    z = jnp.maximum((t - mu) / jnp.sqrt(var + 1e-5) * g_ref[...] + bb_ref[...],
                    0.0)
    hs2_ref[...] = jnp.dot(z, ws_ref[...], preferred_element_type=_f32) + b_ref[...]
    hn2_ref[...] = jnp.dot(z, wn_ref[...], preferred_element_type=_f32)


def _tc_fin(hs_ref, acc_ref, degp_ref, o_ref):
    deg = degp_ref[0, :, :1] + degp_ref[1, :, :1]
    a = acc_ref[0:N] + acc_ref[NP:NP + N]
    t = hs_ref[...] + a / jnp.maximum(deg, 1.0)
    m = jnp.max(t, axis=1, keepdims=True)
    s = t - m
    o_ref[...] = s - jnp.log(jnp.sum(jnp.exp(s), axis=1, keepdims=True))


_nd = jax.ShapeDtypeStruct((N, D), _f32)
_tc_in_call = pl.pallas_call(_tc_in, out_shape=[_nd, _nd])
_tc_mid_call = pl.pallas_call(_tc_mid, out_shape=[_nd, _nd])
_tc_fin_call = pl.pallas_call(_tc_fin, out_shape=_nd)


def kernel(x, edge_index, order_attn, W_self1, W_neigh1, b1, bn1_g, bn1_b,
           W_self2, W_neigh2, b2, bn2_g, bn2_b, W_self3, W_neigh3, b3):
    src = edge_index[0]
    dst = edge_index[1]
    sd2 = jnp.stack([src.reshape(NW * NCHA, CHA), dst.reshape(NW * NCHA, CHA)],
                    axis=1)
    dst2 = dst.reshape(NW * NCHD, CHD)
    z_rows = jnp.zeros((RPT, D), _f32)
    ones_h = jnp.ones((CHD, D), _f32)

    hs1, hn1 = _tc_in_call(x, W_self1, W_neigh1, b1.reshape(1, D))
    degp = _deg(dst2, z_rows, ones_h).reshape(NC, NP, D)[:, :N, :8]
    acc1 = _agg(hn1, sd2, z_rows)
    hs2, hn2 = _tc_mid_call(hs1, acc1, degp, bn1_g.reshape(1, D),
                            bn1_b.reshape(1, D), W_self2, W_neigh2,
                            b2.reshape(1, D))
    acc2 = _agg(hn2, sd2, z_rows)
    hs3, hn3 = _tc_mid_call(hs2, acc2, degp, bn2_g.reshape(1, D),
                            bn2_b.reshape(1, D), W_self3, W_neigh3,
                            b3.reshape(1, D))
    acc3 = _agg(hn3, sd2, z_rows)
    return _tc_fin_call(hs3, acc3, degp)


# speedup vs baseline: 10.6433x; 1.0288x over previous
import functools

import jax
import jax.numpy as jnp
from jax import lax
from jax.experimental import pallas as pl
from jax.experimental.pallas import tpu as pltpu
from jax.experimental.pallas import tpu_sc as plsc

N = 10000
D = 128
E = 320000

NC = 2
NS = 16
NW = NC * NS
EPW = E // NW
CHA = 50
NCHA = EPW // CHA
SLOTS = 5
CHD = 125
NCHD = EPW // CHD
KS = 4
NP = 10240
RPT = NP // NS

_mesh = plsc.VectorSubcoreMesh(core_axis_name="c", subcore_axis_name="s")

_f32 = jnp.float32

_sc_agg = ([pltpu.VMEM((2, 2, CHA), jnp.int32) for _ in range(SLOTS)]
           + [pltpu.VMEM((CHA, D), _f32) for _ in range(SLOTS)]
           + [pltpu.VMEM_SHARED((NP, D), _f32)]
           + [pltpu.SemaphoreType.DMA] * (3 * SLOTS))


@functools.partial(pl.kernel, mesh=_mesh,
                   out_type=jax.ShapeDtypeStruct((NC * NP, D), _f32),
                   scratch_types=_sc_agg)
def _agg(table, sd2, z_rows, acc_out, *rest):
    isl = list(rest[:SLOTS])
    bufs = list(rest[SLOTS:2 * SLOTS])
    acc_sh = rest[2 * SLOTS]
    isem = list(rest[2 * SLOTS + 1:2 * SLOTS + 1 + SLOTS])
    gsem = list(rest[2 * SLOTS + 1 + SLOTS:2 * SLOTS + 1 + 2 * SLOTS])
    ssem = list(rest[2 * SLOTS + 1 + 2 * SLOTS:])
    cid = lax.axis_index("c")
    sid = lax.axis_index("s")
    wid = sid * NC + cid
    row0 = pl.multiple_of(sid * RPT, 8)
    crow = pl.multiple_of(wid * NCHA, 8)
    pltpu.sync_copy(z_rows, acc_sh.at[pl.ds(row0, RPT)])
    plsc.subcore_barrier()

    def idx_load(k, c, p):
        pltpu.async_copy(sd2.at[crow + c], isl[k].at[p], isem[k])

    def idx_wait(k):
        pltpu.make_async_copy(sd2.at[crow], isl[k].at[0], isem[k]).wait()

    def gather(k, p):
        pltpu.async_copy(table.at[isl[k].at[p, 0]], bufs[k], gsem[k])

    def gather_wait(k, p):
        pltpu.make_async_copy(table.at[isl[k].at[p, 0]], bufs[k],
                              gsem[k]).wait()

    def scatter(k, p):
        pltpu.async_copy(bufs[k], acc_sh.at[isl[k].at[p, 1]], ssem[k],
                         add=True)

    def scatter_wait(k):
        pltpu.make_async_copy(bufs[k], acc_sh.at[isl[k].at[0, 1]],
                              ssem[k]).wait()

    for k in range(SLOTS):
        idx_load(k, k, 0)
    for k in range(SLOTS):
        idx_wait(k)
        gather(k, 0)
    for k in range(SLOTS):
        gather_wait(k, 0)
        scatter(k, 0)
        idx_load(k, jnp.minimum(SLOTS + k, NCHA - 1), 1)

    def round_body(r, carry):
        c0 = r * SLOTS
        p = r % 2
        for k in range(SLOTS):
            scatter_wait(k)
            idx_wait(k)
            gather(k, p)
        for k in range(SLOTS):
            gather_wait(k, p)
            scatter(k, p)
            idx_load(k, jnp.minimum(c0 + SLOTS + k, NCHA - 1), 1 - p)
        return carry

    lax.fori_loop(1, NCHA // SLOTS, round_body, 0)
    for k in range(SLOTS):
        scatter_wait(k)
        idx_wait(k)
    plsc.subcore_barrier()
    out_row0 = pl.multiple_of(cid * NP + sid * RPT, 8)
    pltpu.sync_copy(acc_sh.at[pl.ds(row0, RPT)], acc_out.at[pl.ds(out_row0, RPT)])


_sc_deg = [
    pltpu.VMEM((NCHD, CHD), jnp.int32),
    pltpu.VMEM((CHD, D), _f32),
    pltpu.VMEM_SHARED((NP, D), _f32),
] + [pltpu.SemaphoreType.DMA] * KS


@functools.partial(pl.kernel, mesh=_mesh,
                   out_type=jax.ShapeDtypeStruct((NC * NP, D), _f32),
                   scratch_types=_sc_deg)
def _deg(dst2, z_rows, ones_h, deg_out, dst_v, ones_v, deg_sh, *ss):
    cid = lax.axis_index("c")
    sid = lax.axis_index("s")
    wid = sid * NC + cid
    row0 = pl.multiple_of(sid * RPT, 8)
    crow = pl.multiple_of(wid * NCHD, 8)
    pltpu.sync_copy(z_rows, deg_sh.at[pl.ds(row0, RPT)])
    pltpu.sync_copy(ones_h, ones_v)
    pltpu.sync_copy(dst2.at[pl.ds(crow, NCHD)], dst_v)
    plsc.subcore_barrier()

    for k in range(KS):
        pltpu.async_copy(ones_v, deg_sh.at[dst_v.at[k]], ss[k], add=True)

    def body(r, carry):
        c0 = (r + 1) * KS
        for k in range(KS):
            pltpu.make_async_copy(ones_v, deg_sh.at[dst_v.at[0]], ss[k]).wait()
            pltpu.async_copy(ones_v, deg_sh.at[dst_v.at[c0 + k]], ss[k],
                             add=True)
        return carry

    lax.fori_loop(0, NCHD // KS - 1, body, 0)
    for k in range(KS):
        pltpu.make_async_copy(ones_v, deg_sh.at[dst_v.at[0]], ss[k]).wait()
    plsc.subcore_barrier()
    out_row0 = pl.multiple_of(cid * NP + sid * RPT, 8)
    pltpu.sync_copy(deg_sh.at[pl.ds(row0, RPT)], deg_out.at[pl.ds(out_row0, RPT)])


def _tc_in(x_ref, ws_ref, wn_ref, b_ref, hs_ref, hn_ref):
    x = x_ref[...]
    hs_ref[...] = jnp.dot(x, ws_ref[...], preferred_element_type=_f32) + b_ref[...]
    hn_ref[...] = jnp.dot(x, wn_ref[...], preferred_element_type=_f32)


def _tc_mid(hs_ref, acc_ref, degp_ref, g_ref, bb_ref, ws_ref, wn_ref, b_ref,
            hs2_ref, hn2_ref):
    deg = degp_ref[0, :, :1] + degp_ref[1, :, :1]
    a = acc_ref[0:N] + acc_ref[NP:NP + N]
    t = hs_ref[...] + a / jnp.maximum(deg, 1.0)
    mu = jnp.mean(t, axis=0, keepdims=True)
    var = jnp.mean((t - mu) ** 2, axis=0, keepdims=True)
    z = jnp.maximum((t - mu) / jnp.sqrt(var + 1e-5) * g_ref[...] + bb_ref[...],
                    0.0)
    hs2_ref[...] = jnp.dot(z, ws_ref[...], preferred_element_type=_f32) + b_ref[...]
    hn2_ref[...] = jnp.dot(z, wn_ref[...], preferred_element_type=_f32)


def _tc_fin(hs_ref, acc_ref, degp_ref, o_ref):
    deg = degp_ref[0, :, :1] + degp_ref[1, :, :1]
    a = acc_ref[0:N] + acc_ref[NP:NP + N]
    t = hs_ref[...] + a / jnp.maximum(deg, 1.0)
    m = jnp.max(t, axis=1, keepdims=True)
    s = t - m
    o_ref[...] = s - jnp.log(jnp.sum(jnp.exp(s), axis=1, keepdims=True))


_nd = jax.ShapeDtypeStruct((N, D), _f32)
_tc_in_call = pl.pallas_call(_tc_in, out_shape=[_nd, _nd])
_tc_mid_call = pl.pallas_call(_tc_mid, out_shape=[_nd, _nd])
_tc_fin_call = pl.pallas_call(_tc_fin, out_shape=_nd)


def kernel(x, edge_index, order_attn, W_self1, W_neigh1, b1, bn1_g, bn1_b,
           W_self2, W_neigh2, b2, bn2_g, bn2_b, W_self3, W_neigh3, b3):
    src = edge_index[0]
    dst = edge_index[1]
    sd2 = jnp.stack([src.reshape(NW * NCHA, CHA), dst.reshape(NW * NCHA, CHA)],
                    axis=1)
    dst2 = dst.reshape(NW * NCHD, CHD)
    z_rows = jnp.zeros((RPT, D), _f32)
    ones_h = jnp.ones((CHD, D), _f32)

    hs1, hn1 = _tc_in_call(x, W_self1, W_neigh1, b1.reshape(1, D))
    degp = _deg(dst2, z_rows, ones_h).reshape(NC, NP, D)[:, :N, :8]
    acc1 = _agg(hn1, sd2, z_rows)
    hs2, hn2 = _tc_mid_call(hs1, acc1, degp, bn1_g.reshape(1, D),
                            bn1_b.reshape(1, D), W_self2, W_neigh2,
                            b2.reshape(1, D))
    acc2 = _agg(hn2, sd2, z_rows)
    hs3, hn3 = _tc_mid_call(hs2, acc2, degp, bn2_g.reshape(1, D),
                            bn2_b.reshape(1, D), W_self3, W_neigh3,
                            b3.reshape(1, D))
    acc3 = _agg(hn3, sd2, z_rows)
    return _tc_fin_call(hs3, acc3, degp)


# speedup vs baseline: 10.6583x; 1.0014x over previous
import functools

import jax
import jax.numpy as jnp
from jax import lax
from jax.experimental import pallas as pl
from jax.experimental.pallas import tpu as pltpu
from jax.experimental.pallas import tpu_sc as plsc

N = 10000
D = 128
E = 320000

NC = 2
NS = 16
NW = NC * NS
EPW = E // NW
CHA = 50
NCHA = EPW // CHA
SLOTS = 5
CHD = 125
NCHD = EPW // CHD
KS = 4
NP = 10240
RPT = NP // NS

_mesh = plsc.VectorSubcoreMesh(core_axis_name="c", subcore_axis_name="s")

_f32 = jnp.float32

_sc_agg = ([pltpu.VMEM((2, 2, CHA), jnp.int32) for _ in range(SLOTS)]
           + [pltpu.VMEM((CHA, D), _f32) for _ in range(SLOTS)]
           + [pltpu.VMEM_SHARED((NP, D), _f32)]
           + [pltpu.SemaphoreType.DMA] * (3 * SLOTS))


@functools.partial(pl.kernel, mesh=_mesh,
                   out_type=jax.ShapeDtypeStruct((NC * NP, D), _f32),
                   scratch_types=_sc_agg)
def _agg(table, sd2, z_rows, acc_out, *rest):
    isl = list(rest[:SLOTS])
    bufs = list(rest[SLOTS:2 * SLOTS])
    acc_sh = rest[2 * SLOTS]
    isem = list(rest[2 * SLOTS + 1:2 * SLOTS + 1 + SLOTS])
    gsem = list(rest[2 * SLOTS + 1 + SLOTS:2 * SLOTS + 1 + 2 * SLOTS])
    ssem = list(rest[2 * SLOTS + 1 + 2 * SLOTS:])
    cid = lax.axis_index("c")
    sid = lax.axis_index("s")
    wid = sid * NC + cid
    row0 = pl.multiple_of(sid * RPT, 8)
    crow = pl.multiple_of(wid * NCHA, 8)
    pltpu.sync_copy(z_rows, acc_sh.at[pl.ds(row0, RPT)])
    plsc.subcore_barrier()

    def idx_load(k, c, p):
        pltpu.async_copy(sd2.at[crow + c], isl[k].at[p], isem[k])

    def idx_wait(k):
        pltpu.make_async_copy(sd2.at[crow], isl[k].at[0], isem[k]).wait()

    def gather(k, p):
        pltpu.async_copy(table.at[isl[k].at[p, 0]], bufs[k], gsem[k])

    def gather_wait(k, p):
        pltpu.make_async_copy(table.at[isl[k].at[p, 0]], bufs[k],
                              gsem[k]).wait()

    def scatter(k, p):
        pltpu.async_copy(bufs[k], acc_sh.at[isl[k].at[p, 1]], ssem[k],
                         add=True)

    def scatter_wait(k):
        pltpu.make_async_copy(bufs[k], acc_sh.at[isl[k].at[0, 1]],
                              ssem[k]).wait()

    for k in range(SLOTS):
        idx_load(k, k, 0)
    for k in range(SLOTS):
        idx_wait(k)
        gather(k, 0)
    for k in range(SLOTS):
        gather_wait(k, 0)
        scatter(k, 0)
        idx_load(k, jnp.minimum(SLOTS + k, NCHA - 1), 1)

    def round_body(r, carry):
        c0 = r * SLOTS
        p = r % 2
        for k in range(SLOTS):
            scatter_wait(k)
            idx_wait(k)
            gather(k, p)
        for k in range(SLOTS):
            gather_wait(k, p)
            scatter(k, p)
            idx_load(k, jnp.minimum(c0 + SLOTS + k, NCHA - 1), 1 - p)
        return carry

    lax.fori_loop(1, NCHA // SLOTS, round_body, 0)
    for k in range(SLOTS):
        scatter_wait(k)
        idx_wait(k)
    plsc.subcore_barrier()
    out_row0 = pl.multiple_of(cid * NP + sid * RPT, 8)
    pltpu.sync_copy(acc_sh.at[pl.ds(row0, RPT)], acc_out.at[pl.ds(out_row0, RPT)])


_sc_deg = [
    pltpu.VMEM((NCHD, CHD), jnp.int32),
    pltpu.VMEM((CHD, D), _f32),
    pltpu.VMEM_SHARED((NP, D), _f32),
] + [pltpu.SemaphoreType.DMA] * KS


@functools.partial(pl.kernel, mesh=_mesh,
                   out_type=jax.ShapeDtypeStruct((NC * NP, D), _f32),
                   scratch_types=_sc_deg)
def _deg(dst2, z_rows, ones_h, deg_out, dst_v, ones_v, deg_sh, *ss):
    cid = lax.axis_index("c")
    sid = lax.axis_index("s")
    wid = sid * NC + cid
    row0 = pl.multiple_of(sid * RPT, 8)
    crow = pl.multiple_of(wid * NCHD, 8)
    pltpu.sync_copy(z_rows, deg_sh.at[pl.ds(row0, RPT)])
    pltpu.sync_copy(ones_h, ones_v)
    pltpu.sync_copy(dst2.at[pl.ds(crow, NCHD)], dst_v)
    plsc.subcore_barrier()

    for k in range(KS):
        pltpu.async_copy(ones_v, deg_sh.at[dst_v.at[k]], ss[k], add=True)

    def body(r, carry):
        c0 = (r + 1) * KS
        for k in range(KS):
            pltpu.make_async_copy(ones_v, deg_sh.at[dst_v.at[0]], ss[k]).wait()
            pltpu.async_copy(ones_v, deg_sh.at[dst_v.at[c0 + k]], ss[k],
                             add=True)
        return carry

    lax.fori_loop(0, NCHD // KS - 1, body, 0)
    for k in range(KS):
        pltpu.make_async_copy(ones_v, deg_sh.at[dst_v.at[0]], ss[k]).wait()
    plsc.subcore_barrier()
    out_row0 = pl.multiple_of(cid * NP + sid * RPT, 8)
    pltpu.sync_copy(deg_sh.at[pl.ds(row0, RPT)], deg_out.at[pl.ds(out_row0, RPT)])


def _tc_in(x_ref, ws_ref, wn_ref, b_ref, hs_ref, hn_ref):
    x = x_ref[...]
    hs_ref[...] = jnp.dot(x, ws_ref[...], preferred_element_type=_f32) + b_ref[...]
    hn_ref[...] = jnp.dot(x, wn_ref[...], preferred_element_type=_f32)


def _tc_mid(hs_ref, acc_ref, degp_ref, g_ref, bb_ref, ws_ref, wn_ref, b_ref,
            hs2_ref, hn2_ref):
    deg = degp_ref[0, :, :1] + degp_ref[1, :, :1]
    a = acc_ref[0:N] + acc_ref[NP:NP + N]
    t = hs_ref[...] + a / jnp.maximum(deg, 1.0)
    mu = jnp.mean(t, axis=0, keepdims=True)
    var = jnp.mean((t - mu) ** 2, axis=0, keepdims=True)
    z = jnp.maximum((t - mu) / jnp.sqrt(var + 1e-5) * g_ref[...] + bb_ref[...],
                    0.0)
    hs2_ref[...] = jnp.dot(z, ws_ref[...], preferred_element_type=_f32) + b_ref[...]
    hn2_ref[...] = jnp.dot(z, wn_ref[...], preferred_element_type=_f32)


def _tc_fin(hs_ref, acc_ref, degp_ref, o_ref):
    deg = degp_ref[0, :, :1] + degp_ref[1, :, :1]
    a = acc_ref[0:N] + acc_ref[NP:NP + N]
    t = hs_ref[...] + a / jnp.maximum(deg, 1.0)
    m = jnp.max(t, axis=1, keepdims=True)
    s = t - m
    o_ref[...] = s - jnp.log(jnp.sum(jnp.exp(s), axis=1, keepdims=True))


_nd = jax.ShapeDtypeStruct((N, D), _f32)
_tc_in_call = pl.pallas_call(_tc_in, out_shape=[_nd, _nd])
_tc_mid_call = pl.pallas_call(_tc_mid, out_shape=[_nd, _nd])
_tc_fin_call = pl.pallas_call(_tc_fin, out_shape=_nd)


def kernel(x, edge_index, order_attn, W_self1, W_neigh1, b1, bn1_g, bn1_b,
           W_self2, W_neigh2, b2, bn2_g, bn2_b, W_self3, W_neigh3, b3):
    src = edge_index[0]
    dst = edge_index[1]
    sd2 = jnp.stack([src.reshape(NW * NCHA, CHA), dst.reshape(NW * NCHA, CHA)],
                    axis=1)
    dst2 = dst.reshape(NW * NCHD, CHD)
    z_rows = jnp.zeros((RPT, D), _f32)
    ones_h = jnp.ones((CHD, D), _f32)

    degp = _deg(dst2, z_rows, ones_h).reshape(NC, NP, D)[:, :N, :8]
    hs1, hn1 = _tc_in_call(x, W_self1, W_neigh1, b1.reshape(1, D))
    acc1 = _agg(hn1, sd2, z_rows)
    hs2, hn2 = _tc_mid_call(hs1, acc1, degp, bn1_g.reshape(1, D),
                            bn1_b.reshape(1, D), W_self2, W_neigh2,
                            b2.reshape(1, D))
    acc2 = _agg(hn2, sd2, z_rows)
    hs3, hn3 = _tc_mid_call(hs2, acc2, degp, bn2_g.reshape(1, D),
                            bn2_b.reshape(1, D), W_self3, W_neigh3,
                            b3.reshape(1, D))
    acc3 = _agg(hn3, sd2, z_rows)
    return _tc_fin_call(hs3, acc3, degp)
